# baseline (device time: 15230 ns/iter reference)
import jax
import jax.numpy as jnp
from jax import lax
from jax.experimental import pallas as pl
from jax.experimental.pallas import tpu as pltpu

EPS = 1e-5
N_CHUNKS = 8
LAG = 3


def kernel(x, gamma):
    m, n = x.shape
    n_global = 2 * n
    rows = m // 128
    mc = m // N_CHUNKS
    rc = rows // N_CHUNKS
    gamma2d = pltpu.with_memory_space_constraint(
        gamma.reshape(1, n), pltpu.MemorySpace.HBM)

    def body(x_hbm, g_hbm, out_ref, x_vmem, g_vmem, send_buf, recv_buf,
             in_sems, g_sem, send_sems, recv_sems):
        my_x = lax.axis_index("x")
        my_y = lax.axis_index("y")
        partner = (my_x, 1 - my_y)

        barrier = pltpu.get_barrier_semaphore()
        pl.semaphore_signal(
            barrier, inc=1, device_id=partner,
            device_id_type=pl.DeviceIdType.MESH,
        )

        g_copy = pltpu.make_async_copy(g_hbm, g_vmem, g_sem)
        g_copy.start()
        in_copies = []
        for c in range(N_CHUNKS):
            cp = pltpu.make_async_copy(
                x_hbm.at[pl.ds(c * mc, mc), :],
                x_vmem.at[pl.ds(c * mc, mc), :],
                in_sems.at[c],
            )
            cp.start()
            in_copies.append(cp)
        g_copy.wait()

        rdmas = []

        def phase1(c):
            in_copies[c].wait()
            xc3 = x_vmem[pl.ds(c * mc, mc), :].reshape(rc, 128, n)
            send_buf[pl.ds(c * rc, rc), :] = jnp.sum(xc3 * xc3, axis=2)
            rdma = pltpu.make_async_remote_copy(
                src_ref=send_buf.at[pl.ds(c * rc, rc), :],
                dst_ref=recv_buf.at[pl.ds(c * rc, rc), :],
                send_sem=send_sems.at[c],
                recv_sem=recv_sems.at[c],
                device_id=partner,
                device_id_type=pl.DeviceIdType.MESH,
            )
            rdma.start()
            rdmas.append(rdma)

        def phase2(c):
            rdmas[c].wait_recv()
            total = (send_buf[c * rc:(c + 1) * rc, :]
                     + recv_buf[c * rc:(c + 1) * rc, :])
            invc = lax.rsqrt(total / n_global + EPS)
            xc3 = x_vmem[pl.ds(c * mc, mc), :].reshape(rc, 128, n)
            outc = (xc3 * invc[:, :, None]).reshape(mc, n) * g_vmem[:, :]
            out_ref[pl.ds(c * mc, mc), :] = outc.astype(out_ref.dtype)

        for c in range(N_CHUNKS):
            phase1(c)
            if c >= LAG:
                phase2(c - LAG)
        for c in range(N_CHUNKS - LAG, N_CHUNKS):
            phase2(c)

        for c in range(N_CHUNKS):
            rdmas[c].wait_send()
        pl.semaphore_wait(barrier, 1)

    return pl.pallas_call(
        body,
        out_shape=jax.ShapeDtypeStruct((m, n), jnp.bfloat16),
        in_specs=[
            pl.BlockSpec(memory_space=pl.ANY),
            pl.BlockSpec(memory_space=pl.ANY),
        ],
        out_specs=pl.BlockSpec(memory_space=pltpu.VMEM),
        scratch_shapes=[
            pltpu.VMEM((m, n), jnp.float32),
            pltpu.VMEM((1, n), jnp.float32),
            pltpu.VMEM((rows, 128), jnp.float32),
            pltpu.VMEM((rows, 128), jnp.float32),
            pltpu.SemaphoreType.DMA((N_CHUNKS,)),
            pltpu.SemaphoreType.DMA,
            pltpu.SemaphoreType.DMA((N_CHUNKS,)),
            pltpu.SemaphoreType.DMA((N_CHUNKS,)),
        ],
        compiler_params=pltpu.CompilerParams(collective_id=0),
    )(x, gamma2d)


# device time: 14247 ns/iter; 1.0690x vs baseline; 1.0690x over previous
import jax
import jax.numpy as jnp
from jax import lax
from jax.experimental import pallas as pl
from jax.experimental.pallas import tpu as pltpu

EPS = 1e-5
N_CHUNKS = 8
LAG = 3


def kernel(x, gamma):
    m, n = x.shape
    n_global = 2 * n
    rows = m // 128
    mc = m // N_CHUNKS
    rc = rows // N_CHUNKS
    gamma2d = pltpu.with_memory_space_constraint(
        gamma.reshape(1, n), pltpu.MemorySpace.HBM)

    def body(x_hbm, g_hbm, out_ref, x_vmem, g_vmem, send_buf, recv_buf,
             in_sems, g_sem, send_sems, recv_sems):
        my_x = lax.axis_index("x")
        my_y = lax.axis_index("y")
        partner = (my_x, 1 - my_y)

        barrier = pltpu.get_barrier_semaphore()
        pl.semaphore_signal(
            barrier, inc=1, device_id=partner,
            device_id_type=pl.DeviceIdType.MESH,
        )

        g_copy = pltpu.make_async_copy(g_hbm, g_vmem, g_sem)
        g_copy.start()
        in_copies = []
        for c in range(N_CHUNKS):
            cp = pltpu.make_async_copy(
                x_hbm.at[pl.ds(c * mc, mc), :],
                x_vmem.at[pl.ds(c * mc, mc), :],
                in_sems.at[c],
            )
            cp.start()
            in_copies.append(cp)

        rdmas = []
        g_waited = []

        def get_g():
            if not g_waited:
                g_copy.wait()
                g_waited.append(True)
            return g_vmem[:, :]

        def phase1(c):
            in_copies[c].wait()
            xc3 = x_vmem[pl.ds(c * mc, mc), :].reshape(rc, 128, n)
            send_buf[pl.ds(c * rc, rc), :] = jnp.sum(xc3 * xc3, axis=2)
            rdma = pltpu.make_async_remote_copy(
                src_ref=send_buf.at[pl.ds(c * rc, rc), :],
                dst_ref=recv_buf.at[pl.ds(c * rc, rc), :],
                send_sem=send_sems.at[c],
                recv_sem=recv_sems.at[c],
                device_id=partner,
                device_id_type=pl.DeviceIdType.MESH,
            )
            rdma.start()
            rdmas.append(rdma)

        def phase2(c):
            rdmas[c].wait_recv()
            total = (send_buf[c * rc:(c + 1) * rc, :]
                     + recv_buf[c * rc:(c + 1) * rc, :])
            invc = lax.rsqrt(total / n_global + EPS)
            xc3 = x_vmem[pl.ds(c * mc, mc), :].reshape(rc, 128, n)
            outc = (xc3 * invc[:, :, None]).reshape(mc, n) * get_g()
            out_ref[pl.ds(c * mc, mc), :] = outc.astype(out_ref.dtype)

        for c in range(N_CHUNKS):
            phase1(c)
            if c >= LAG:
                phase2(c - LAG)
        for c in range(N_CHUNKS - LAG, N_CHUNKS):
            phase2(c)

        for c in range(N_CHUNKS):
            rdmas[c].wait_send()
        pl.semaphore_wait(barrier, 1)

    return pl.pallas_call(
        body,
        out_shape=jax.ShapeDtypeStruct((m, n), jnp.bfloat16),
        in_specs=[
            pl.BlockSpec(memory_space=pl.ANY),
            pl.BlockSpec(memory_space=pl.ANY),
        ],
        out_specs=pl.BlockSpec(memory_space=pltpu.VMEM),
        scratch_shapes=[
            pltpu.VMEM((m, n), jnp.float32),
            pltpu.VMEM((1, n), jnp.float32),
            pltpu.VMEM((rows, 128), jnp.float32),
            pltpu.VMEM((rows, 128), jnp.float32),
            pltpu.SemaphoreType.DMA((N_CHUNKS,)),
            pltpu.SemaphoreType.DMA,
            pltpu.SemaphoreType.DMA((N_CHUNKS,)),
            pltpu.SemaphoreType.DMA((N_CHUNKS,)),
        ],
        compiler_params=pltpu.CompilerParams(collective_id=0),
    )(x, gamma2d)


# device time: 13545 ns/iter; 1.1244x vs baseline; 1.0518x over previous
import jax
import jax.numpy as jnp
from jax import lax
from jax.experimental import pallas as pl
from jax.experimental.pallas import tpu as pltpu

EPS = 1e-5
N_CHUNKS = 4
LAG = 2


def kernel(x, gamma):
    m, n = x.shape
    n_global = 2 * n
    rows = m // 128
    mc = m // N_CHUNKS
    rc = rows // N_CHUNKS
    gamma2d = pltpu.with_memory_space_constraint(
        gamma.reshape(1, n), pltpu.MemorySpace.HBM)

    def body(x_hbm, g_hbm, out_ref, x_vmem, g_vmem, send_buf, recv_buf,
             in_sems, g_sem, send_sems, recv_sems):
        my_x = lax.axis_index("x")
        my_y = lax.axis_index("y")
        partner = (my_x, 1 - my_y)

        barrier = pltpu.get_barrier_semaphore()
        pl.semaphore_signal(
            barrier, inc=1, device_id=partner,
            device_id_type=pl.DeviceIdType.MESH,
        )

        g_copy = pltpu.make_async_copy(g_hbm, g_vmem, g_sem)
        g_copy.start()
        in_copies = []
        for c in range(N_CHUNKS):
            cp = pltpu.make_async_copy(
                x_hbm.at[pl.ds(c * mc, mc), :],
                x_vmem.at[pl.ds(c * mc, mc), :],
                in_sems.at[c],
            )
            cp.start()
            in_copies.append(cp)

        rdmas = []
        g_waited = []

        def get_g():
            if not g_waited:
                g_copy.wait()
                g_waited.append(True)
            return g_vmem[:, :]

        def phase1(c):
            in_copies[c].wait()
            xc3 = x_vmem[pl.ds(c * mc, mc), :].reshape(rc, 128, n)
            send_buf[pl.ds(c * rc, rc), :] = jnp.sum(xc3 * xc3, axis=2)
            rdma = pltpu.make_async_remote_copy(
                src_ref=send_buf.at[pl.ds(c * rc, rc), :],
                dst_ref=recv_buf.at[pl.ds(c * rc, rc), :],
                send_sem=send_sems.at[c],
                recv_sem=recv_sems.at[c],
                device_id=partner,
                device_id_type=pl.DeviceIdType.MESH,
            )
            rdma.start()
            rdmas.append(rdma)

        def phase2(c):
            rdmas[c].wait_recv()
            total = (send_buf[c * rc:(c + 1) * rc, :]
                     + recv_buf[c * rc:(c + 1) * rc, :])
            invc = lax.rsqrt(total / n_global + EPS)
            xc3 = x_vmem[pl.ds(c * mc, mc), :].reshape(rc, 128, n)
            outc = (xc3 * invc[:, :, None]).reshape(mc, n) * get_g()
            out_ref[pl.ds(c * mc, mc), :] = outc.astype(out_ref.dtype)

        for c in range(N_CHUNKS):
            phase1(c)
            if c >= LAG:
                phase2(c - LAG)
        for c in range(N_CHUNKS - LAG, N_CHUNKS):
            phase2(c)

        for c in range(N_CHUNKS):
            rdmas[c].wait_send()
        pl.semaphore_wait(barrier, 1)

    return pl.pallas_call(
        body,
        out_shape=jax.ShapeDtypeStruct((m, n), jnp.bfloat16),
        in_specs=[
            pl.BlockSpec(memory_space=pl.ANY),
            pl.BlockSpec(memory_space=pl.ANY),
        ],
        out_specs=pl.BlockSpec(memory_space=pltpu.VMEM),
        scratch_shapes=[
            pltpu.VMEM((m, n), jnp.float32),
            pltpu.VMEM((1, n), jnp.float32),
            pltpu.VMEM((rows, 128), jnp.float32),
            pltpu.VMEM((rows, 128), jnp.float32),
            pltpu.SemaphoreType.DMA((N_CHUNKS,)),
            pltpu.SemaphoreType.DMA,
            pltpu.SemaphoreType.DMA((N_CHUNKS,)),
            pltpu.SemaphoreType.DMA((N_CHUNKS,)),
        ],
        compiler_params=pltpu.CompilerParams(collective_id=0),
    )(x, gamma2d)
